# Initial kernel scaffold; baseline (speedup 1.0000x reference)
import jax
import jax.numpy as jnp
from jax.experimental import pallas as pl


def kernel(x, W_emb, W_fc, bias):
    raise NotImplementedError("write your pallas kernel here")


# speedup vs baseline: 11.1339x; 11.1339x over previous
import functools

import jax
import jax.numpy as jnp
from jax import lax
from jax.experimental import pallas as pl
from jax.experimental.pallas import tpu as pltpu
from jax.experimental.pallas import tpu_sc as plsc

NC, NS, L = 2, 16, 16
NW = NC * NS
B = 1024
FD, FS = 13, 26
F = FD + FS
K = 16
VOCAB = 1000
FEAT = FD + FS * VOCAB
RPW = B // NW
NIDX = FS * F
NUD = FD * F

_mesh = plsc.VectorSubcoreMesh(
    core_axis_name="c", subcore_axis_name="s", num_cores=NC, num_subcores=NS
)


@functools.partial(
    pl.kernel,
    out_type=jax.ShapeDtypeStruct((B,), jnp.float32),
    mesh=_mesh,
    compiler_params=pltpu.CompilerParams(
        use_tc_tiling_on_sc=False, needs_layout_passes=False
    ),
    scratch_types=[
        pltpu.VMEM((RPW, F), jnp.float32),
        pltpu.VMEM((48,), jnp.int32),
        pltpu.VMEM((64 * L,), jnp.int32),
        pltpu.VMEM((64 * L,), jnp.int32),
        pltpu.VMEM((8, 128), jnp.int32),
        pltpu.VMEM((4, 128), jnp.int32),
        pltpu.VMEM((8 * 128, K), jnp.float32),
        pltpu.VMEM((4 * 128, K), jnp.float32),
        pltpu.VMEM((16, K), jnp.float32),
        pltpu.VMEM((48,), jnp.int32),
        pltpu.VMEM((48, K), jnp.float32),
        pltpu.VMEM((RPW,), jnp.float32),
        pltpu.VMEM((L,), jnp.float32),
        pltpu.SemaphoreType.DMA,
        pltpu.SemaphoreType.DMA,
    ],
)
def _ffm_sc(x_hbm, w_hbm, wfc_hbm, bias_hbm, out_hbm,
            xch, otab, jvf, ivm, idx_s, udidx, rows, udv, dbuf,
            fcidx, fcr, zbuf, bv, sem, sem2):
    wid = lax.axis_index("s") * NC + lax.axis_index("c")
    base = wid * RPW
    iota = lax.iota(jnp.int32, L)
    zf = jnp.zeros((L,), jnp.float32)

    pltpu.sync_copy(x_hbm.at[pl.ds(base, RPW)], xch)
    pltpu.sync_copy(bias_hbm, bv)

    for k in range(3):
        n = iota + 16 * k
        val = jnp.where(n < FD, n,
                        jnp.where(n < F, FD + (n - FD) * VOCAB, 0))
        otab[pl.ds(16 * k, 16)] = val

    def _tabs(k, _):
        n = iota + k * 16
        jq = n // F
        ir = n - jq * F
        jvf[pl.ds(k * 16, 16)] = FD + jq
        ivm[pl.ds(k * 16, 16)] = ir * FEAT
        return _
    lax.fori_loop(0, 64, _tabs, 0)

    def _udidx(k, _):
        n = iota + k * 16
        dq = n // F
        ir = n - dq * F
        udidx[k // 8, pl.ds((k % 8) * 16, 16)] = jnp.where(
            n < NUD, ir * FEAT + dq, 0)
        return _
    lax.fori_loop(0, 32, _udidx, 0)
    for q in range(4):
        pltpu.async_copy(w_hbm.at[udidx.at[q]],
                         udv.at[pl.ds(q * 128, 128)], sem).wait()

    ic = jnp.minimum(iota, FD - 1)

    def _drow(j, _):
        def _dk(k, acc):
            kf = jnp.full((L,), k, jnp.int32)
            a = plsc.load_gather(udv, [j * F + ic, kf])
            b = plsc.load_gather(udv, [ic * F + j, kf])
            return acc + a * b
        accd = lax.fori_loop(0, K, _dk, zf)
        dbuf[j] = jnp.where(iota < FD, accd, 0.0)
        return _
    lax.fori_loop(0, FD, _drow, 0)

    def _row(r, _):
        rfull = jnp.full((L,), r, jnp.int32)

        for k in range(3):
            fcidx[pl.ds(16 * k, 16)] = otab[pl.ds(16 * k, 16)]
        g1 = plsc.load_gather(xch, [rfull, jnp.minimum(iota + FD, F - 1)])
        g2 = plsc.load_gather(xch, [rfull, jnp.minimum(iota + FD + 16, F - 1)])
        plsc.addupdate_scatter(fcidx, [iota + FD], g1.astype(jnp.int32))
        plsc.addupdate_scatter(fcidx, [iota + FD + 16], g2.astype(jnp.int32),
                               mask=iota < (F - FD - 16))

        def _bidx(k, _):
            jv = jvf[pl.ds(k * 16, 16)]
            iv = ivm[pl.ds(k * 16, 16)]
            feat = plsc.load_gather(fcidx, [jv])
            idx_s[k // 8, pl.ds((k % 8) * 16, 16)] = iv + feat
            return _
        lax.fori_loop(0, 64, _bidx, 0)

        handles = [
            pltpu.async_copy(w_hbm.at[idx_s.at[q]],
                             rows.at[pl.ds(q * 128, 128)], sem)
            for q in range(8)
        ]
        hfc = pltpu.async_copy(wfc_hbm.at[fcidx], fcr, sem2)
        for h in handles:
            h.wait()
        hfc.wait()

        xd = plsc.load_gather(xch, [rfull, ic])
        xd = jnp.where(iota < FD, xd, 0.0)

        def _dd(j, acc):
            bx = plsc.load_gather(xch, [rfull, jnp.full((L,), j, jnp.int32)])
            return acc + bx * (dbuf[j] * xd)
        accv = lax.fori_loop(0, FD, _dd, zf)

        def _dso(d, acc):
            def _dsi(sp, a):
                t = rows[sp * F + d] * udv[d * F + FD + sp]
                w = jnp.where(sp < FS - 1, 2.0, 1.0)
                return a + w * t
            am = lax.fori_loop(0, FS, _dsi, zf)
            bx = plsc.load_gather(xch, [rfull, jnp.full((L,), d, jnp.int32)])
            return acc + bx * am
        accv = lax.fori_loop(0, FD, _dso, accv)

        def _sso(jp, acc):
            def _ssi(ip, a):
                return a + rows[jp * F + FD + ip] * rows[ip * F + FD + jp]
            inner = lax.fori_loop(0, jp, _ssi, zf)
            w = jnp.where(jp < FS - 1, 2.0, 1.0)
            dg = rows[jp * F + FD + jp]
            dw = jnp.where(jp < FS - 1, 1.0, 0.0)
            return acc + w * inner + dw * (dg * dg)
        accv = lax.fori_loop(0, FS, _sso, accv)

        f0 = plsc.load_gather(fcr, [iota, jnp.zeros((L,), jnp.int32)])
        f1 = plsc.load_gather(fcr, [iota + 16, jnp.zeros((L,), jnp.int32)])
        f2 = plsc.load_gather(fcr, [jnp.minimum(iota + 32, 47),
                                    jnp.zeros((L,), jnp.int32)])
        f2 = jnp.where(iota < F - 32, f2, 0.0)
        accv = accv + f0 + f1 + f2 + jnp.where(iota < 1, bv[...], 0.0)

        z = jnp.sum(accv)
        plsc.store_scatter(zbuf, [rfull], jnp.full((L,), z), mask=iota < 1)
        return _
    lax.fori_loop(0, RPW, _row, 0)

    for k in range(2):
        zv = zbuf[pl.ds(k * 16, 16)]
        zbuf[pl.ds(k * 16, 16)] = 1.0 / (1.0 + jnp.exp(-zv))
    pltpu.sync_copy(zbuf, out_hbm.at[pl.ds(base, RPW)])


def kernel(x, W_emb, W_fc, bias):
    w_flat = W_emb.reshape(F * FEAT, K)
    wfc16 = jnp.pad(W_fc, ((0, 0), (0, K - 1)))
    bias16 = jnp.broadcast_to(bias, (L,))
    out = _ffm_sc(x, w_flat, wfc16, bias16)
    return out.reshape(B, 1)


# speedup vs baseline: 36.2126x; 3.2525x over previous
import functools

import jax
import jax.numpy as jnp
from jax import lax
from jax.experimental import pallas as pl
from jax.experimental.pallas import tpu as pltpu
from jax.experimental.pallas import tpu_sc as plsc

NC, NS, L = 2, 16, 16
NW = NC * NS
B = 1024
FD, FS = 13, 26
F = FD + FS
K = 16
VOCAB = 1000
FEAT = FD + FS * VOCAB
FEATP = 26112
RPW = B // NW
NM = FS * 5
CH = 2176

_mesh = plsc.VectorSubcoreMesh(
    core_axis_name="c", subcore_axis_name="s", num_cores=NC, num_subcores=NS
)


def _tc_transpose_body(in_ref, out_ref):
    out_ref[...] = in_ref[...].T


@functools.partial(
    pl.kernel,
    out_type=jax.ShapeDtypeStruct((B,), jnp.float32),
    mesh=_mesh,
    compiler_params=pltpu.CompilerParams(
        use_tc_tiling_on_sc=False, needs_layout_passes=False
    ),
    scratch_types=[
        pltpu.VMEM((RPW, F), jnp.float32),
        pltpu.VMEM((FEAT,), jnp.float32),
        pltpu.VMEM((48,), jnp.int32),
        pltpu.VMEM((48,), jnp.int32),
        pltpu.VMEM((65,), jnp.int32),
        pltpu.VMEM((65,), jnp.int32),
        pltpu.VMEM((65,), jnp.int32),
        pltpu.VMEM((NM, 128), jnp.float32),
        pltpu.VMEM((65, 128), jnp.float32),
        pltpu.VMEM((16, K), jnp.float32),
        pltpu.VMEM((RPW,), jnp.float32),
        pltpu.VMEM((L,), jnp.float32),
        pltpu.SemaphoreType.DMA,
        pltpu.SemaphoreType.DMA,
    ],
)
def _ffm_sc(x_hbm, w5_hbm, wfc_hbm, bias_hbm, out_hbm,
            xch, wfcv, otab, fcidx, idxa, idxb, udidx, rows_v, udv, dbuf,
            zbuf, bv, sem, sem2):
    wid = lax.axis_index("s") * NC + lax.axis_index("c")
    base = wid * RPW
    iota = lax.iota(jnp.int32, L)
    zf = jnp.zeros((L,), jnp.float32)

    pltpu.sync_copy(x_hbm.at[pl.ds(base, RPW)], xch)
    pltpu.sync_copy(bias_hbm, bv)
    pltpu.sync_copy(wfc_hbm, wfcv)

    for k in range(3):
        n = iota + 16 * k
        otab[pl.ds(16 * k, 16)] = jnp.where(
            n < FD, n, jnp.where(n < F, FD + (n - FD) * VOCAB, 0))

    for v in range(5):
        m = iota + 16 * v
        mq = m // 5
        val = (m - 5 * mq) * FEATP + mq
        plsc.store_scatter(udidx, [jnp.minimum(m, 64)], val, mask=m < 65)
    pltpu.async_copy(w5_hbm.at[udidx], udv, sem).wait()

    ic = jnp.minimum(iota, FD - 1)
    icq = ic // 8
    icc = (ic - 8 * icq) * 16

    def _drow(j, _):
        jq = j // 8
        jc = (j - 8 * jq) * 16

        def _dk(k, acc):
            a = plsc.load_gather(udv, [j * 5 + icq, icc + k])
            b = plsc.load_gather(udv, [ic * 5 + jq, jnp.full((L,), jc + k,
                                                             jnp.int32)])
            return acc + a * b
        accd = lax.fori_loop(0, K, _dk, zf)
        dbuf[j] = jnp.where(iota < FD, accd, 0.0)
        return _
    lax.fori_loop(0, FD, _drow, 0)

    def _row(r, _):
        rfull = jnp.full((L,), r, jnp.int32)

        for k in range(3):
            fcidx[pl.ds(16 * k, 16)] = otab[pl.ds(16 * k, 16)]
        g1 = plsc.load_gather(xch, [rfull, jnp.minimum(iota + FD, F - 1)])
        g2 = plsc.load_gather(xch, [rfull, jnp.minimum(iota + FD + 16, F - 1)])
        plsc.addupdate_scatter(fcidx, [iota + FD], g1.astype(jnp.int32))
        plsc.addupdate_scatter(fcidx, [iota + FD + 16], g2.astype(jnp.int32),
                               mask=iota < (F - FD - 16))

        for v in range(9):
            m = iota + 16 * v
            mq = jnp.minimum(m // 5, FS)
            feat = plsc.load_gather(fcidx, [FD + mq])
            val = (m - 5 * (m // 5)) * FEATP + feat
            plsc.store_scatter(idxa, [jnp.minimum(m, 64)], val, mask=m < 65)
            plsc.store_scatter(idxb, [jnp.clip(m - 65, 0, 64)], val,
                               mask=jnp.logical_and(m >= 65, m < NM))
        ha = pltpu.async_copy(w5_hbm.at[idxa], rows_v.at[pl.ds(0, 65)], sem)
        hb = pltpu.async_copy(w5_hbm.at[idxb], rows_v.at[pl.ds(65, 65)], sem2)
        ha.wait()
        hb.wait()

        xd = plsc.load_gather(xch, [rfull, ic])
        xd = jnp.where(iota < FD, xd, 0.0)

        def _dd(j, acc):
            bx = plsc.load_gather(xch, [rfull, jnp.full((L,), j, jnp.int32)])
            return acc + bx * (dbuf[j] * xd)
        accv = lax.fori_loop(0, FD, _dd, zf)

        def _dso(d, acc):
            dq = d // 8
            dc = (d - 8 * dq) * 16

            def _dsi(sp, a):
                i = FD + sp
                iq = i // 8
                icl = (i - 8 * iq) * 16
                t = (rows_v[sp * 5 + dq, pl.ds(dc, 16)]
                     * udv[d * 5 + iq, pl.ds(icl, 16)])
                w = jnp.where(sp < FS - 1, 2.0, 1.0)
                return a + w * t
            am = lax.fori_loop(0, FS, _dsi, zf)
            bx = plsc.load_gather(xch, [rfull, jnp.full((L,), d, jnp.int32)])
            return acc + bx * am
        accv = lax.fori_loop(0, FD, _dso, accv)

        def _sso(jp, acc):
            j = FD + jp
            jq = j // 8
            jc = (j - 8 * jq) * 16

            def _ssi(ip, a):
                i = FD + ip
                iq = i // 8
                icl = (i - 8 * iq) * 16
                va = rows_v[jp * 5 + iq, pl.ds(icl, 16)]
                vb = rows_v[ip * 5 + jq, pl.ds(jc, 16)]
                return a + va * vb
            inner = lax.fori_loop(0, jp, _ssi, zf)
            w = jnp.where(jp < FS - 1, 2.0, 1.0)
            dg = rows_v[jp * 5 + jq, pl.ds(jc, 16)]
            dw = jnp.where(jp < FS - 1, 1.0, 0.0)
            return acc + w * inner + dw * (dg * dg)
        accv = lax.fori_loop(0, FS, _sso, accv)

        f0 = plsc.load_gather(wfcv, [plsc.load_gather(fcidx, [iota])])
        f1 = plsc.load_gather(wfcv, [plsc.load_gather(fcidx, [iota + 16])])
        f2 = plsc.load_gather(wfcv, [plsc.load_gather(fcidx, [iota + 32])])
        f2 = jnp.where(iota < F - 32, f2, 0.0)
        accv = accv + f0 + f1 + f2 + jnp.where(iota < 1, bv[...], 0.0)

        z = jnp.sum(accv)
        plsc.store_scatter(zbuf, [rfull], jnp.full((L,), z), mask=iota < 1)
        return _
    lax.fori_loop(0, RPW, _row, 0)

    for k in range(2):
        zv = zbuf[pl.ds(k * 16, 16)]
        zbuf[pl.ds(k * 16, 16)] = 1.0 / (1.0 + jnp.exp(-zv))
    pltpu.sync_copy(zbuf, out_hbm.at[pl.ds(base, RPW)])


def kernel(x, W_emb, W_fc, bias):
    wn2d = jnp.transpose(W_emb, (0, 2, 1)).reshape(F * K, FEAT)
    t5 = pl.pallas_call(
        _tc_transpose_body,
        grid=(FEATP // CH, 5),
        in_specs=[pl.BlockSpec((128, CH), lambda c, q: (q, c))],
        out_specs=pl.BlockSpec((CH, 128), lambda c, q: (q * (FEATP // CH) + c,
                                                        0)),
        out_shape=jax.ShapeDtypeStruct((5 * FEATP, 128), jnp.float32),
    )(wn2d)
    wfc_flat = W_fc.reshape(FEAT)
    bias16 = jnp.broadcast_to(bias, (L,))
    out = _ffm_sc(x, t5, wfc_flat, bias16)
    return out.reshape(B, 1)
